# baseline (speedup 1.0000x reference)
import jax
import jax.numpy as jnp
from jax import lax
from jax.experimental import pallas as pl
from jax.experimental.pallas import tpu as pltpu, tpu_sc as plsc

B, L, D = 4096, 200, 16
NC, NS = 2, 16
NW = NC * NS
NB = L // 8
SLOTS = 8


def _embed_body(xq, xp, xt, xi, wq, wp, wt, wi, out,
                idxq, idxb, rowsq, outb, tp, tt, ti,
                isem, qsem, wsem):
    wid = lax.axis_index("s") * NC + lax.axis_index("c")
    xsml = (xp, xt, xi)
    smalls = ((0, tp, wp), (1, tt, wt), (2, ti, wi))

    for _, tbl, w in smalls:
        pltpu.sync_copy(w, tbl)

    def fire_idx(tr, pp):
        pltpu.async_copy(xq.at[tr, wid], idxq.at[pp], isem.at[pp])
        for f in range(3):
            pltpu.async_copy(xsml[f].at[tr, wid], idxb.at[pp, f],
                             isem.at[pp])

    def wait_idx(tr, pp):
        pltpu.make_async_copy(xq.at[tr, wid], idxq.at[pp],
                              isem.at[pp]).wait()
        for f in range(3):
            pltpu.make_async_copy(xsml[f].at[tr, wid], idxb.at[pp, f],
                                  isem.at[pp]).wait()

    def qgather(pp):
        return pltpu.make_async_copy(wq.at[idxq.at[pp]], rowsq.at[pp],
                                     qsem.at[pp])

    def wblock(l, s):
        return pltpu.make_async_copy(outb.at[s], out.at[l, :, wid],
                                     wsem.at[s])

    fire_idx(0, 0)
    wait_idx(0, 0)
    pltpu.async_copy(wq.at[idxq.at[0]], rowsq.at[0], qsem.at[0])
    fire_idx(1, 1)

    @pl.loop(0, NB)
    def band(tr):
        p = tr % 2
        np_ = 1 - p

        @pl.when(tr + 1 < NB)
        def _fire_next_band():
            wait_idx(tr + 1, np_)
            pltpu.async_copy(wq.at[idxq.at[np_]], rowsq.at[np_],
                             qsem.at[np_])

        qgather(p).wait()
        rq = rowsq.at[p]

        for sub in range(8):
            l = tr * 8 + sub

            @pl.when(l >= SLOTS)
            def _recycle_slot():
                wblock(l, sub).wait()

            @plsc.parallel_loop(0, 8, unroll=2)
            def bgrp(g):
                bvec = lax.iota(jnp.int32, 16) + (sub * 128 + g * 16)
                for c in range(16):
                    cvec = jnp.full((16,), c, jnp.int32)
                    vals = plsc.load_gather(rq, [bvec, cvec])
                    outb[sub, c // 8, c % 8, pl.ds(g * 16, 16)] = vals
                for f, tbl, _ in smalls:
                    idx16 = idxb[p, f, sub, pl.ds(g * 16, 16)]
                    for c in range(16):
                        cvec = jnp.full((16,), c, jnp.int32)
                        vals = plsc.load_gather(tbl, [idx16, cvec])
                        cc = (f + 1) * 16 + c
                        outb[sub, cc // 8, cc % 8, pl.ds(g * 16, 16)] = vals

            pltpu.async_copy(outb.at[sub], out.at[l, :, wid], wsem.at[sub])

        @pl.when(tr + 2 < NB)
        def _prefetch_idx():
            fire_idx(tr + 2, p)

    for s in range(SLOTS):
        wblock(L - SLOTS + s, (L - SLOTS + s) % SLOTS).wait()


def _wt_body(wt_ref, out_ref):
    out_ref[...] = wt_ref[...].T


def _w_row_major(W):
    V, VP = W.shape[0], 512 * ((W.shape[0] + 511) // 512)
    return pl.pallas_call(
        _wt_body,
        grid=(VP // 512,),
        in_specs=[pl.BlockSpec((D, 512), lambda i: (0, i))],
        out_specs=pl.BlockSpec((512, D), lambda i: (i, 0)),
        out_shape=jax.ShapeDtypeStruct((V, D), jnp.float32),
    )(W.T)


@jax.jit
def kernel(x_question_id, x_part, x_tag, x_interaction,
           W_question_id, W_part, W_tag, W_interaction):
    def view4(x):
        return x.T.reshape(NB, 8, NW, 128).transpose(0, 2, 1, 3)

    xqv = view4(x_question_id).reshape(NB, NW, 1024)
    xs = [view4(x) for x in (x_part, x_tag, x_interaction)]
    mesh = plsc.VectorSubcoreMesh(core_axis_name="c", subcore_axis_name="s",
                                  num_cores=NC, num_subcores=NS)
    out5 = pl.kernel(
        _embed_body,
        out_type=jax.ShapeDtypeStruct((L, 8, NW, 8, 128), jnp.float32),
        mesh=mesh,
        scratch_types=[
            pltpu.VMEM((2, 1024), jnp.int32),
            pltpu.VMEM((2, 3, 8, 128), jnp.int32),
            pltpu.VMEM((2, 1024, D), jnp.float32),
            pltpu.VMEM((SLOTS, 8, 8, 128), jnp.float32),
            pltpu.VMEM((9, D), jnp.float32),
            pltpu.VMEM((189, D), jnp.float32),
            pltpu.VMEM((3, D), jnp.float32),
            pltpu.SemaphoreType.DMA((2,)),
            pltpu.SemaphoreType.DMA((2,)),
            pltpu.SemaphoreType.DMA((SLOTS,)),
        ],
        compiler_params=pltpu.CompilerParams(use_tc_tiling_on_sc=False,
                                             needs_layout_passes=False),
    )(xqv, xs[0], xs[1], xs[2],
      _w_row_major(W_question_id), W_part, W_tag, W_interaction)
    return out5.transpose(2, 4, 0, 1, 3).reshape(B, L, 4 * D)


# speedup vs baseline: 2.1047x; 2.1047x over previous
import jax
import jax.numpy as jnp
from jax import lax
from jax.experimental import pallas as pl
from jax.experimental.pallas import tpu as pltpu, tpu_sc as plsc

B, L, D = 4096, 200, 16
NC, NS = 2, 16
NW = NC * NS
NB = L // 8
SLOTS = 8


def _embed_body(xq, xp, xt, xi, wq, wp, wt, wi, out,
                idxq, idxb, rowsq, outb, tp, tt, ti,
                isem, qsem, wsem):
    wid = lax.axis_index("s") * NC + lax.axis_index("c")
    xsml = (xp, xt, xi)
    smalls = ((0, tp, wp), (1, tt, wt), (2, ti, wi))

    for _, tbl, w in smalls:
        pltpu.sync_copy(w, tbl)

    def fire_idx(tr, pp):
        pltpu.async_copy(xq.at[tr, wid], idxq.at[pp], isem.at[pp])
        for f in range(3):
            pltpu.async_copy(xsml[f].at[tr, wid], idxb.at[pp, f],
                             isem.at[pp])

    def wait_idx(tr, pp):
        pltpu.make_async_copy(xq.at[tr, wid], idxq.at[pp],
                              isem.at[pp]).wait()
        for f in range(3):
            pltpu.make_async_copy(xsml[f].at[tr, wid], idxb.at[pp, f],
                                  isem.at[pp]).wait()

    def qgather(pp):
        return pltpu.make_async_copy(wq.at[idxq.at[pp]], rowsq.at[pp],
                                     qsem.at[pp])

    def wblock(l, s):
        return pltpu.make_async_copy(outb.at[s], out.at[l, :, wid],
                                     wsem.at[s])

    fire_idx(0, 0)
    wait_idx(0, 0)
    pltpu.async_copy(wq.at[idxq.at[0]], rowsq.at[0], qsem.at[0])
    fire_idx(1, 1)

    @pl.loop(0, NB)
    def band(tr):
        p = tr % 2
        np_ = 1 - p

        @pl.when(tr + 1 < NB)
        def _fire_next_band():
            wait_idx(tr + 1, np_)
            pltpu.async_copy(wq.at[idxq.at[np_]], rowsq.at[np_],
                             qsem.at[np_])

        qgather(p).wait()
        rq = rowsq.at[p]

        for sub in range(8):
            l = tr * 8 + sub

            @pl.when(l >= SLOTS)
            def _recycle_slot():
                wblock(l, sub).wait()

            @plsc.parallel_loop(0, 8, unroll=2)
            def bgrp(g):
                bvec = lax.iota(jnp.int32, 16) + (sub * 128 + g * 16)
                for c in range(16):
                    cvec = jnp.full((16,), c, jnp.int32)
                    vals = plsc.load_gather(rq, [bvec, cvec])
                    outb[sub, c // 8, c % 8, pl.ds(g * 16, 16)] = vals
                for f, tbl, _ in smalls:
                    idx16 = idxb[p, f, sub, pl.ds(g * 16, 16)]
                    for c in range(16):
                        cvec = jnp.full((16,), c, jnp.int32)
                        vals = plsc.load_gather(tbl, [idx16, cvec])
                        cc = (f + 1) * 16 + c
                        outb[sub, cc // 8, cc % 8, pl.ds(g * 16, 16)] = vals

            pltpu.async_copy(outb.at[sub], out.at[l, :, wid], wsem.at[sub])

        @pl.when(tr + 2 < NB)
        def _prefetch_idx():
            fire_idx(tr + 2, p)

    for s in range(SLOTS):
        wblock(L - SLOTS + s, (L - SLOTS + s) % SLOTS).wait()


@jax.jit
def kernel(x_question_id, x_part, x_tag, x_interaction,
           W_question_id, W_part, W_tag, W_interaction):
    def view4(x):
        return x.T.reshape(NB, 8, NW, 128).transpose(0, 2, 1, 3)

    xqv = view4(x_question_id).reshape(NB, NW, 1024)
    xs = [view4(x) for x in (x_part, x_tag, x_interaction)]
    mesh = plsc.VectorSubcoreMesh(core_axis_name="c", subcore_axis_name="s",
                                  num_cores=NC, num_subcores=NS)
    out5 = pl.kernel(
        _embed_body,
        out_type=jax.ShapeDtypeStruct((L, 8, NW, 8, 128), jnp.float32),
        mesh=mesh,
        scratch_types=[
            pltpu.VMEM((2, 1024), jnp.int32),
            pltpu.VMEM((2, 3, 8, 128), jnp.int32),
            pltpu.VMEM((2, 1024, D), jnp.float32),
            pltpu.VMEM((SLOTS, 8, 8, 128), jnp.float32),
            pltpu.VMEM((9, D), jnp.float32),
            pltpu.VMEM((189, D), jnp.float32),
            pltpu.VMEM((3, D), jnp.float32),
            pltpu.SemaphoreType.DMA((2,)),
            pltpu.SemaphoreType.DMA((2,)),
            pltpu.SemaphoreType.DMA((SLOTS,)),
        ],
        compiler_params=pltpu.CompilerParams(use_tc_tiling_on_sc=False,
                                             needs_layout_passes=False),
    )(xqv, xs[0], xs[1], xs[2],
      W_question_id, W_part, W_tag, W_interaction)
    return out5.transpose(2, 4, 0, 1, 3).reshape(B, L, 4 * D)
